# Initial kernel scaffold; baseline (speedup 1.0000x reference)
import jax
import jax.numpy as jnp
from jax.experimental import pallas as pl


def kernel(x, edge_index, W1, b1, W2, b2):
    raise NotImplementedError("write your pallas kernel here")


# speedup vs baseline: 15.1032x; 15.1032x over previous
import functools

import jax
import jax.numpy as jnp
from jax import lax
from jax.experimental import pallas as pl
from jax.experimental.pallas import tpu as pltpu
from jax.experimental.pallas import tpu_sc as plsc

N_NODES = 10000
IN_CH = 256
HID_CH = 32
N_EDGES = 160000

NC, NS = 2, 16
NW = NC * NS
NPAD = 10240
ROWS_PER_TILE = NPAD // NS
K = 128
CHUNKS = 40
E_TILE = K * CHUNKS
EPAD = NW * E_TILE
DUMMY = N_NODES
DEG_W = 16

_mesh = plsc.VectorSubcoreMesh(core_axis_name="c", subcore_axis_name="s")
_sc_params = pltpu.CompilerParams(use_tc_tiling_on_sc=False)


@functools.partial(
    pl.kernel,
    mesh=_mesh,
    out_type=jax.ShapeDtypeStruct((NC * NPAD, DEG_W), jnp.float32),
    scratch_types=[
        pltpu.VMEM((CHUNKS, K), jnp.int32),
        pltpu.VMEM((K, DEG_W), jnp.float32),
        pltpu.VMEM_SHARED((NPAD, DEG_W), jnp.float32),
    ],
    compiler_params=_sc_params,
)
def _sc_deg(dst_hbm, zeros1_hbm, ones_hbm, out_hbm, dstv, onesv, deg_sh):
    c = lax.axis_index("c")
    s = lax.axis_index("s")
    wid = s * NC + c
    r0 = s * ROWS_PER_TILE
    pltpu.sync_copy(zeros1_hbm.at[pl.ds(r0, ROWS_PER_TILE)],
                    deg_sh.at[pl.ds(r0, ROWS_PER_TILE)])
    pltpu.sync_copy(ones_hbm, onesv)
    pltpu.sync_copy(dst_hbm.at[pl.ds(wid * CHUNKS, CHUNKS)], dstv)
    plsc.subcore_barrier()

    def body(j, carry):
        pltpu.sync_copy(onesv, deg_sh.at[dstv.at[j]], add=True)
        return carry

    lax.fori_loop(0, CHUNKS, body, 0)
    plsc.subcore_barrier()
    pltpu.sync_copy(deg_sh.at[pl.ds(r0, ROWS_PER_TILE)],
                    out_hbm.at[pl.ds(c * NPAD + r0, ROWS_PER_TILE)])


@functools.partial(
    pl.kernel,
    mesh=_mesh,
    out_type=jax.ShapeDtypeStruct((NC * NPAD, HID_CH), jnp.float32),
    scratch_types=[
        pltpu.VMEM((CHUNKS, K), jnp.int32),
        pltpu.VMEM((CHUNKS, K), jnp.int32),
        pltpu.VMEM((K, HID_CH), jnp.float32),
        pltpu.VMEM_SHARED((NPAD, HID_CH), jnp.float32),
        pltpu.SemaphoreType.DMA,
    ],
    compiler_params=_sc_params,
)
def _sc_agg(src_hbm, dst_hbm, feat_hbm, zeros_hbm, out_hbm,
            srcv, dstv, rows, agg_sh, sem):
    c = lax.axis_index("c")
    s = lax.axis_index("s")
    wid = s * NC + c
    r0 = s * ROWS_PER_TILE
    pltpu.sync_copy(zeros_hbm.at[pl.ds(r0, ROWS_PER_TILE)],
                    agg_sh.at[pl.ds(r0, ROWS_PER_TILE)])
    pltpu.sync_copy(src_hbm.at[pl.ds(wid * CHUNKS, CHUNKS)], srcv)
    pltpu.sync_copy(dst_hbm.at[pl.ds(wid * CHUNKS, CHUNKS)], dstv)
    plsc.subcore_barrier()

    def body(j, carry):
        pltpu.async_copy(feat_hbm.at[srcv.at[j]], rows, sem).wait()
        pltpu.sync_copy(rows, agg_sh.at[dstv.at[j]], add=True)
        return carry

    lax.fori_loop(0, CHUNKS, body, 0)
    plsc.subcore_barrier()
    pltpu.sync_copy(agg_sh.at[pl.ds(r0, ROWS_PER_TILE)],
                    out_hbm.at[pl.ds(c * NPAD + r0, ROWS_PER_TILE)])


_RB = 1280


def _tc1_body(x_ref, w_ref, d0_ref, d1_ref, hs_ref, dinv_ref):
    deg = d0_ref[...] + d1_ref[...] + 1.0
    dinv = lax.rsqrt(deg)
    h = jnp.dot(x_ref[...], w_ref[...], preferred_element_type=jnp.float32)
    hs_ref[...] = h * dinv
    dinv_ref[...] = dinv


def _tc2_body(a0_ref, a1_ref, hs_ref, dinv_ref, b1_ref, zs_ref):
    dinv = dinv_ref[...]
    t = (a0_ref[...] + a1_ref[...] + hs_ref[...]) * dinv + b1_ref[...]
    zs_ref[...] = jnp.maximum(t, 0.0) * dinv


def _tc3_body(a0_ref, a1_ref, zs_ref, dinv_ref, w2_ref, b2_ref, out_ref):
    t = a0_ref[...] + a1_ref[...] + zs_ref[...]
    g = jnp.dot(t, w2_ref[...], preferred_element_type=jnp.float32)
    out_ref[...] = g * dinv_ref[...] + b2_ref[...]


def _row_spec(width):
    return pl.BlockSpec((_RB, width), lambda i: (i, 0))


def _full_spec(shape):
    return pl.BlockSpec(shape, lambda i: (0, 0))


def kernel(x, edge_index, W1, b1, W2, b2):
    ei = edge_index.astype(jnp.int32)
    pad = jnp.full((EPAD - N_EDGES,), DUMMY, dtype=jnp.int32)
    src = jnp.concatenate([ei[0], pad]).reshape(NW * CHUNKS, K)
    dst = jnp.concatenate([ei[1], pad]).reshape(NW * CHUNKS, K)

    x_pad = jnp.zeros((NPAD, IN_CH), jnp.float32).at[:N_NODES].set(x)
    zeros_w = jnp.zeros((NPAD, HID_CH), jnp.float32)
    zeros_d = jnp.zeros((NPAD, DEG_W), jnp.float32)
    ones_k = jnp.ones((K, DEG_W), jnp.float32)
    w2p = jnp.zeros((HID_CH, 128), jnp.float32).at[:, :2].set(W2)
    b2p = jnp.zeros((1, 128), jnp.float32).at[0, :2].set(b2)
    b1r = b1.reshape(1, HID_CH)

    deg2 = _sc_deg(dst, zeros_d, ones_k)
    deg0 = deg2[:NPAD, :1]
    deg1 = deg2[NPAD:, :1]

    hs, dinv = pl.pallas_call(
        _tc1_body,
        grid=(NPAD // _RB,),
        in_specs=[_row_spec(IN_CH), _full_spec((IN_CH, HID_CH)),
                  _row_spec(1), _row_spec(1)],
        out_specs=[_row_spec(HID_CH), _row_spec(1)],
        out_shape=[jax.ShapeDtypeStruct((NPAD, HID_CH), jnp.float32),
                   jax.ShapeDtypeStruct((NPAD, 1), jnp.float32)],
    )(x_pad, W1, deg0, deg1)

    agg1 = _sc_agg(src, dst, hs, zeros_w)

    zs = pl.pallas_call(
        _tc2_body,
        grid=(NPAD // _RB,),
        in_specs=[_row_spec(HID_CH), _row_spec(HID_CH), _row_spec(HID_CH),
                  _row_spec(1), _full_spec((1, HID_CH))],
        out_specs=_row_spec(HID_CH),
        out_shape=jax.ShapeDtypeStruct((NPAD, HID_CH), jnp.float32),
    )(agg1[:NPAD], agg1[NPAD:], hs, dinv, b1r)

    agg2 = _sc_agg(src, dst, zs, zeros_w)

    outp = pl.pallas_call(
        _tc3_body,
        grid=(NPAD // _RB,),
        in_specs=[_row_spec(HID_CH), _row_spec(HID_CH), _row_spec(HID_CH),
                  _row_spec(1), _full_spec((HID_CH, 128)),
                  _full_spec((1, 128))],
        out_specs=_row_spec(128),
        out_shape=jax.ShapeDtypeStruct((NPAD, 128), jnp.float32),
    )(agg2[:NPAD], agg2[NPAD:], zs, dinv, w2p, b2p)

    return outp[:N_NODES, :2]


# speedup vs baseline: 17.2219x; 1.1403x over previous
import functools

import jax
import jax.numpy as jnp
from jax import lax
from jax.experimental import pallas as pl
from jax.experimental.pallas import tpu as pltpu
from jax.experimental.pallas import tpu_sc as plsc

N_NODES = 10000
IN_CH = 256
HID_CH = 32
N_EDGES = 160000

NC, NS = 2, 16
NW = NC * NS
NPAD = 10240
ROWS_PER_TILE = NPAD // NS
K = 128
CHUNKS = 40
E_TILE = K * CHUNKS
EPAD = NW * E_TILE
DUMMY = N_NODES
DEG_W = 16

_mesh = plsc.VectorSubcoreMesh(core_axis_name="c", subcore_axis_name="s")
_sc_params = pltpu.CompilerParams(use_tc_tiling_on_sc=False)


@functools.partial(
    pl.kernel,
    mesh=_mesh,
    out_type=jax.ShapeDtypeStruct((NC * NPAD, DEG_W), jnp.float32),
    scratch_types=[
        pltpu.VMEM((CHUNKS, K), jnp.int32),
        pltpu.VMEM((K, DEG_W), jnp.float32),
        pltpu.VMEM_SHARED((NPAD, DEG_W), jnp.float32),
    ],
    compiler_params=_sc_params,
)
def _sc_deg(dst_hbm, zeros1_hbm, ones_hbm, out_hbm, dstv, onesv, deg_sh):
    c = lax.axis_index("c")
    s = lax.axis_index("s")
    wid = s * NC + c
    r0 = s * ROWS_PER_TILE
    pltpu.sync_copy(zeros1_hbm.at[pl.ds(r0, ROWS_PER_TILE)],
                    deg_sh.at[pl.ds(r0, ROWS_PER_TILE)])
    pltpu.sync_copy(ones_hbm, onesv)
    pltpu.sync_copy(dst_hbm.at[pl.ds(wid * CHUNKS, CHUNKS)], dstv)
    plsc.subcore_barrier()

    def body(j, carry):
        pltpu.sync_copy(onesv, deg_sh.at[dstv.at[j]], add=True)
        return carry

    lax.fori_loop(0, CHUNKS, body, 0)
    plsc.subcore_barrier()
    pltpu.sync_copy(deg_sh.at[pl.ds(r0, ROWS_PER_TILE)],
                    out_hbm.at[pl.ds(c * NPAD + r0, ROWS_PER_TILE)])


@functools.partial(
    pl.kernel,
    mesh=_mesh,
    out_type=jax.ShapeDtypeStruct((NC * NPAD, HID_CH), jnp.float32),
    scratch_types=[
        pltpu.VMEM((CHUNKS, K), jnp.int32),
        pltpu.VMEM((CHUNKS, K), jnp.int32),
        pltpu.VMEM((2, K, HID_CH), jnp.float32),
        pltpu.VMEM_SHARED((NPAD, HID_CH), jnp.float32),
        pltpu.SemaphoreType.DMA,
        pltpu.SemaphoreType.DMA,
    ],
    compiler_params=_sc_params,
)
def _sc_agg(src_hbm, dst_hbm, feat_hbm, zeros_hbm, out_hbm,
            srcv, dstv, rows, agg_sh, sem0, sem1):
    c = lax.axis_index("c")
    s = lax.axis_index("s")
    wid = s * NC + c
    r0 = s * ROWS_PER_TILE
    pltpu.sync_copy(zeros_hbm.at[pl.ds(r0, ROWS_PER_TILE)],
                    agg_sh.at[pl.ds(r0, ROWS_PER_TILE)])
    pltpu.sync_copy(src_hbm.at[pl.ds(wid * CHUNKS, CHUNKS)], srcv)
    pltpu.sync_copy(dst_hbm.at[pl.ds(wid * CHUNKS, CHUNKS)], dstv)
    plsc.subcore_barrier()

    sems = (sem0, sem1)
    pltpu.async_copy(feat_hbm.at[srcv.at[0]], rows.at[0], sem0)
    pltpu.async_copy(feat_hbm.at[srcv.at[1]], rows.at[1], sem1)

    def body(j, carry):
        for b in range(2):
            jj = j + b
            pltpu.make_async_copy(feat_hbm.at[srcv.at[jj]], rows.at[b],
                                  sems[b]).wait()
            pltpu.sync_copy(rows.at[b], agg_sh.at[dstv.at[jj]], add=True)

            @pl.when(jj + 2 < CHUNKS)
            def _():
                pltpu.async_copy(feat_hbm.at[srcv.at[jj + 2]], rows.at[b],
                                 sems[b])
        return carry

    lax.fori_loop(0, CHUNKS // 2, lambda i, c: body(i * 2, c), 0)
    plsc.subcore_barrier()
    pltpu.sync_copy(agg_sh.at[pl.ds(r0, ROWS_PER_TILE)],
                    out_hbm.at[pl.ds(c * NPAD + r0, ROWS_PER_TILE)])


_RB = 1280


def _tc1_body(x_ref, w_ref, d0_ref, d1_ref, hs_ref, dinv_ref):
    deg = d0_ref[...] + d1_ref[...] + 1.0
    dinv = lax.rsqrt(deg)
    h = jnp.dot(x_ref[...], w_ref[...], preferred_element_type=jnp.float32)
    hs_ref[...] = h * dinv
    dinv_ref[...] = dinv


def _tc2_body(a0_ref, a1_ref, hs_ref, dinv_ref, b1_ref, zs_ref):
    dinv = dinv_ref[...]
    t = (a0_ref[...] + a1_ref[...] + hs_ref[...]) * dinv + b1_ref[...]
    zs_ref[...] = jnp.maximum(t, 0.0) * dinv


def _tc3_body(a0_ref, a1_ref, zs_ref, dinv_ref, w2_ref, b2_ref, out_ref):
    t = a0_ref[...] + a1_ref[...] + zs_ref[...]
    g = jnp.dot(t, w2_ref[...], preferred_element_type=jnp.float32)
    out_ref[...] = g * dinv_ref[...] + b2_ref[...]


def _row_spec(width):
    return pl.BlockSpec((_RB, width), lambda i: (i, 0))


def _full_spec(shape):
    return pl.BlockSpec(shape, lambda i: (0, 0))


def kernel(x, edge_index, W1, b1, W2, b2):
    ei = edge_index.astype(jnp.int32)
    pad = jnp.full((EPAD - N_EDGES,), DUMMY, dtype=jnp.int32)
    src = jnp.concatenate([ei[0], pad]).reshape(NW * CHUNKS, K)
    dst = jnp.concatenate([ei[1], pad]).reshape(NW * CHUNKS, K)

    x_pad = jnp.zeros((NPAD, IN_CH), jnp.float32).at[:N_NODES].set(x)
    zeros_w = jnp.zeros((NPAD, HID_CH), jnp.float32)
    zeros_d = jnp.zeros((NPAD, DEG_W), jnp.float32)
    ones_k = jnp.ones((K, DEG_W), jnp.float32)
    w2p = jnp.zeros((HID_CH, 128), jnp.float32).at[:, :2].set(W2)
    b2p = jnp.zeros((1, 128), jnp.float32).at[0, :2].set(b2)
    b1r = b1.reshape(1, HID_CH)

    deg2 = _sc_deg(dst, zeros_d, ones_k)
    deg0 = deg2[:NPAD, :1]
    deg1 = deg2[NPAD:, :1]

    hs, dinv = pl.pallas_call(
        _tc1_body,
        grid=(NPAD // _RB,),
        in_specs=[_row_spec(IN_CH), _full_spec((IN_CH, HID_CH)),
                  _row_spec(1), _row_spec(1)],
        out_specs=[_row_spec(HID_CH), _row_spec(1)],
        out_shape=[jax.ShapeDtypeStruct((NPAD, HID_CH), jnp.float32),
                   jax.ShapeDtypeStruct((NPAD, 1), jnp.float32)],
    )(x_pad, W1, deg0, deg1)

    agg1 = _sc_agg(src, dst, hs, zeros_w)

    zs = pl.pallas_call(
        _tc2_body,
        grid=(NPAD // _RB,),
        in_specs=[_row_spec(HID_CH), _row_spec(HID_CH), _row_spec(HID_CH),
                  _row_spec(1), _full_spec((1, HID_CH))],
        out_specs=_row_spec(HID_CH),
        out_shape=jax.ShapeDtypeStruct((NPAD, HID_CH), jnp.float32),
    )(agg1[:NPAD], agg1[NPAD:], hs, dinv, b1r)

    agg2 = _sc_agg(src, dst, zs, zeros_w)

    outp = pl.pallas_call(
        _tc3_body,
        grid=(NPAD // _RB,),
        in_specs=[_row_spec(HID_CH), _row_spec(HID_CH), _row_spec(HID_CH),
                  _row_spec(1), _full_spec((HID_CH, 128)),
                  _full_spec((1, 128))],
        out_specs=_row_spec(128),
        out_shape=jax.ShapeDtypeStruct((NPAD, 128), jnp.float32),
    )(agg2[:NPAD], agg2[NPAD:], zs, dinv, w2p, b2p)

    return outp[:N_NODES, :2]


# speedup vs baseline: 28.1767x; 1.6361x over previous
import functools

import jax
import jax.numpy as jnp
from jax import lax
from jax.experimental import pallas as pl
from jax.experimental.pallas import tpu as pltpu
from jax.experimental.pallas import tpu_sc as plsc

N = 10000
IN_CH = 256
HID_CH = 32
N_EDGES = 160000

NC, NS = 2, 16
NW = NC * NS
RPT = N // NS
K = 125
CHUNKS = 40
DEG_W = 16
NBUF = 4

_mesh = plsc.VectorSubcoreMesh(core_axis_name="c", subcore_axis_name="s")
_sc_params = pltpu.CompilerParams(use_tc_tiling_on_sc=False)


@functools.partial(
    pl.kernel,
    mesh=_mesh,
    out_type=jax.ShapeDtypeStruct((NC * N, DEG_W), jnp.float32),
    scratch_types=[
        pltpu.VMEM((CHUNKS, K), jnp.int32),
        pltpu.VMEM((K, DEG_W), jnp.float32),
        pltpu.VMEM_SHARED((N, DEG_W), jnp.float32),
        pltpu.SemaphoreType.DMA,
    ],
    compiler_params=_sc_params,
)
def _sc_deg(dst_hbm, zeros_hbm, ones_hbm, out_hbm, dstv, onesv, deg_sh, sem):
    c = lax.axis_index("c")
    s = lax.axis_index("s")
    wid = s * NC + c
    r0 = s * RPT
    pltpu.sync_copy(zeros_hbm.at[pl.ds(r0, RPT)], deg_sh.at[pl.ds(r0, RPT)])
    pltpu.sync_copy(ones_hbm, onesv)
    pltpu.sync_copy(dst_hbm.at[pl.ds(wid * CHUNKS, CHUNKS)], dstv)
    plsc.subcore_barrier()

    def fire(j, carry):
        pltpu.async_copy(onesv, deg_sh.at[dstv.at[j]], sem, add=True)
        return carry

    lax.fori_loop(0, CHUNKS, fire, 0)

    def drain(j, carry):
        pltpu.make_async_copy(onesv, deg_sh.at[dstv.at[0]], sem).wait()
        return carry

    lax.fori_loop(0, CHUNKS, drain, 0)
    plsc.subcore_barrier()
    pltpu.sync_copy(deg_sh.at[pl.ds(r0, RPT)],
                    out_hbm.at[pl.ds(c * N + r0, RPT)])


@functools.partial(
    pl.kernel,
    mesh=_mesh,
    out_type=jax.ShapeDtypeStruct((NC * N, HID_CH), jnp.float32),
    scratch_types=[
        pltpu.VMEM((CHUNKS, K), jnp.int32),
        pltpu.VMEM((CHUNKS, K), jnp.int32),
        pltpu.VMEM((NBUF, K, HID_CH), jnp.float32),
        pltpu.VMEM_SHARED((N, HID_CH), jnp.float32),
    ] + [pltpu.SemaphoreType.DMA] * (2 * NBUF),
    compiler_params=_sc_params,
)
def _sc_agg(src_hbm, dst_hbm, feat_hbm, zeros_hbm, out_hbm,
            srcv, dstv, rows, agg_sh, *sems):
    gsem = sems[:NBUF]
    ssem = sems[NBUF:]
    c = lax.axis_index("c")
    s = lax.axis_index("s")
    wid = s * NC + c
    r0 = s * RPT
    pltpu.sync_copy(zeros_hbm.at[pl.ds(r0, RPT)], agg_sh.at[pl.ds(r0, RPT)])
    pltpu.sync_copy(src_hbm.at[pl.ds(wid * CHUNKS, CHUNKS)], srcv)
    pltpu.sync_copy(dst_hbm.at[pl.ds(wid * CHUNKS, CHUNKS)], dstv)
    plsc.subcore_barrier()

    pltpu.async_copy(feat_hbm.at[srcv.at[0]], rows.at[0], gsem[0])
    pltpu.async_copy(feat_hbm.at[srcv.at[1]], rows.at[1], gsem[1])

    def body(i, carry):
        for b in range(NBUF):
            t = i * NBUF + b
            b2 = (b + 2) % NBUF
            pltpu.make_async_copy(feat_hbm.at[srcv.at[t]], rows.at[b],
                                  gsem[b]).wait()
            pltpu.async_copy(rows.at[b], agg_sh.at[dstv.at[t]], ssem[b],
                             add=True)

            @pl.when(t >= 2)
            def _():
                pltpu.make_async_copy(rows.at[b2], agg_sh.at[dstv.at[t]],
                                      ssem[b2]).wait()

            @pl.when(t + 2 < CHUNKS)
            def _():
                pltpu.async_copy(feat_hbm.at[srcv.at[t + 2]], rows.at[b2],
                                 gsem[b2])
        return carry

    lax.fori_loop(0, CHUNKS // NBUF, body, 0)
    pltpu.make_async_copy(rows.at[2], agg_sh.at[dstv.at[0]], ssem[2]).wait()
    pltpu.make_async_copy(rows.at[3], agg_sh.at[dstv.at[0]], ssem[3]).wait()
    plsc.subcore_barrier()
    pltpu.sync_copy(agg_sh.at[pl.ds(r0, RPT)],
                    out_hbm.at[pl.ds(c * N + r0, RPT)])


_RB = 1000
_GRID_OFF = N // _RB


def _tc1_body(x_ref, w_ref, d0_ref, d1_ref, hs_ref, dinv_ref):
    deg = d0_ref[:, :1] + d1_ref[:, :1] + 1.0
    dinv = lax.rsqrt(deg)
    h = jnp.dot(x_ref[...], w_ref[...], preferred_element_type=jnp.float32)
    hs_ref[...] = h * dinv
    dinv_ref[...] = dinv


def _tc2_body(a0_ref, a1_ref, hs_ref, dinv_ref, b1_ref, zs_ref):
    dinv = dinv_ref[...]
    t = (a0_ref[...] + a1_ref[...] + hs_ref[...]) * dinv + b1_ref[...]
    zs_ref[...] = jnp.maximum(t, 0.0) * dinv


def _tc3_body(a0_ref, a1_ref, zs_ref, dinv_ref, w2_ref, b2_ref, out_ref):
    t = a0_ref[...] + a1_ref[...] + zs_ref[...]
    g = jnp.dot(t, w2_ref[...], preferred_element_type=jnp.float32)
    out_ref[...] = g * dinv_ref[...] + b2_ref[...]


def _row_spec(width):
    return pl.BlockSpec((_RB, width), lambda i: (i, 0))


def _row_spec_hi(width):
    return pl.BlockSpec((_RB, width), lambda i: (i + _GRID_OFF, 0))


def _full_spec(shape):
    return pl.BlockSpec(shape, lambda i: (0, 0))


def kernel(x, edge_index, W1, b1, W2, b2):
    ei = edge_index.astype(jnp.int32)
    src = ei[0].reshape(NW * CHUNKS, K)
    dst = ei[1].reshape(NW * CHUNKS, K)

    zeros_w = jnp.zeros((N, HID_CH), jnp.float32)
    zeros_d = jnp.zeros((N, DEG_W), jnp.float32)
    ones_k = jnp.ones((K, DEG_W), jnp.float32)
    w2p = jnp.zeros((HID_CH, 128), jnp.float32).at[:, :2].set(W2)
    b2p = jnp.zeros((1, 128), jnp.float32).at[0, :2].set(b2)
    b1r = b1.reshape(1, HID_CH)

    deg2 = _sc_deg(dst, zeros_d, ones_k)

    hs, dinv = pl.pallas_call(
        _tc1_body,
        grid=(_GRID_OFF,),
        in_specs=[_row_spec(IN_CH), _full_spec((IN_CH, HID_CH)),
                  _row_spec(DEG_W), _row_spec_hi(DEG_W)],
        out_specs=[_row_spec(HID_CH), _row_spec(1)],
        out_shape=[jax.ShapeDtypeStruct((N, HID_CH), jnp.float32),
                   jax.ShapeDtypeStruct((N, 1), jnp.float32)],
    )(x, W1, deg2, deg2)

    agg1 = _sc_agg(src, dst, hs, zeros_w)

    zs = pl.pallas_call(
        _tc2_body,
        grid=(_GRID_OFF,),
        in_specs=[_row_spec(HID_CH), _row_spec_hi(HID_CH), _row_spec(HID_CH),
                  _row_spec(1), _full_spec((1, HID_CH))],
        out_specs=_row_spec(HID_CH),
        out_shape=jax.ShapeDtypeStruct((N, HID_CH), jnp.float32),
    )(agg1, agg1, hs, dinv, b1r)

    agg2 = _sc_agg(src, dst, zs, zeros_w)

    outp = pl.pallas_call(
        _tc3_body,
        grid=(_GRID_OFF,),
        in_specs=[_row_spec(HID_CH), _row_spec_hi(HID_CH), _row_spec(HID_CH),
                  _row_spec(1), _full_spec((HID_CH, 128)),
                  _full_spec((1, 128))],
        out_specs=_row_spec(128),
        out_shape=jax.ShapeDtypeStruct((N, 128), jnp.float32),
    )(agg2, agg2, zs, dinv, w2p, b2p)

    return outp[:, :2]
